# Initial kernel scaffold; baseline (speedup 1.0000x reference)
import jax
import jax.numpy as jnp
from jax.experimental import pallas as pl


def kernel(inputs, manualrand):
    raise NotImplementedError("write your pallas kernel here")


# speedup vs baseline: 3.0342x; 3.0342x over previous
import functools

import jax
import jax.numpy as jnp
from jax.experimental import pallas as pl
from jax.experimental.pallas import tpu as pltpu

B = 128
V = 100000
C = 2048
NBLK = (V + C - 1) // C

NEG_INF = float("-inf")


def _cumsum_lanes(p, width):
    k = 1
    while k < width:
        shifted = jnp.concatenate(
            [jnp.zeros((p.shape[0], k), p.dtype), p[:, : width - k]], axis=1
        )
        p = p + shifted
        k *= 2
    return p


def _scan_kernel(x_ref, rand_ref, out_ref, lp_ref, carry_ref, acc_ref):
    i = pl.program_id(0)

    @pl.when(i == 0)
    def _init():
        carry_ref[...] = jnp.zeros((B, 1), jnp.float32)
        acc_ref[...] = jnp.zeros((B, 1), jnp.float32)

    x = x_ref[...]
    col = i * C + jax.lax.broadcasted_iota(jnp.int32, (B, C), 1)
    valid = col < V
    p = jnp.where(valid, jnp.exp(x), 0.0)
    cs = _cumsum_lanes(p, C)
    carry_in = carry_ref[...]
    total = carry_in + cs
    rand = rand_ref[...]
    prev_total = jnp.concatenate([carry_in, total[:, : C - 1]], axis=1)
    onehot = jnp.logical_and(total >= rand, prev_total < rand)
    out_ref[...] = jnp.where(onehot, 0.0, NEG_INF)
    acc_ref[...] += jnp.sum(jnp.where(onehot, x, 0.0), axis=1, keepdims=True)
    carry_ref[...] = total[:, C - 1 : C]

    @pl.when(i == NBLK - 1)
    def _fin():
        lp_ref[...] = acc_ref[...]


@jax.jit
def kernel(inputs, manualrand):
    out, lp = pl.pallas_call(
        _scan_kernel,
        grid=(NBLK,),
        in_specs=[
            pl.BlockSpec((B, C), lambda i: (0, i)),
            pl.BlockSpec((B, 1), lambda i: (0, 0)),
        ],
        out_specs=[
            pl.BlockSpec((B, C), lambda i: (0, i)),
            pl.BlockSpec((B, 1), lambda i: (0, 0)),
        ],
        out_shape=[
            jax.ShapeDtypeStruct((B, V), jnp.float32),
            jax.ShapeDtypeStruct((B, 1), jnp.float32),
        ],
        scratch_shapes=[
            pltpu.VMEM((B, 1), jnp.float32),
            pltpu.VMEM((B, 1), jnp.float32),
        ],
    )(inputs, manualrand)
    return out, lp


# speedup vs baseline: 5.7417x; 1.8923x over previous
import jax
import jax.numpy as jnp
from jax.experimental import pallas as pl
from jax.experimental.pallas import tpu as pltpu

B = 128
V = 100000
CS = 1024
NCH = V // CS
TS = V - NCH * CS
CF = 4096
NBLK = (V + CF - 1) // CF

NEG_INF = float("-inf")


def _cumsum_lanes(p, width):
    k = 1
    while k < width:
        shifted = jnp.concatenate(
            [jnp.zeros((p.shape[0], k), p.dtype), p[:, : width - k]], axis=1
        )
        p = p + shifted
        k *= 2
    return p


def _process_chunk(x, base, width, carry, rand, idx, lp):
    p = jnp.exp(x)
    total = carry + _cumsum_lanes(p, width)
    prev_total = jnp.concatenate([carry, total[:, : width - 1]], axis=1)
    onehot = jnp.logical_and(total >= rand, prev_total < rand)
    col = base + jax.lax.broadcasted_iota(jnp.int32, (B, width), 1)
    has = jnp.any(onehot, axis=1, keepdims=True)
    idx_new = jnp.sum(jnp.where(onehot, col, 0), axis=1, keepdims=True)
    idx = jnp.where(has, idx_new, idx)
    lp = lp + jnp.sum(jnp.where(onehot, x, 0.0), axis=1, keepdims=True)
    return total[:, width - 1 :], idx, lp


def _kernel(x_hbm, rand_ref, out_ref, lp_ref, idx_ref, chunk_ref, tail_ref, sem):
    i = pl.program_id(0)
    rand = rand_ref[...]

    @pl.when(i == 0)
    def _search():
        def cond(state):
            c, carry, _, _ = state
            return jnp.logical_and(c < NCH, jnp.logical_not(jnp.all(carry >= rand)))

        def body(state):
            c, carry, idx, lp = state
            copy = pltpu.make_async_copy(
                x_hbm.at[:, pl.ds(c * CS, CS)], chunk_ref, sem
            )
            copy.start()
            copy.wait()
            carry, idx, lp = _process_chunk(
                chunk_ref[...], c * CS, CS, carry, rand, idx, lp
            )
            return c + 1, carry, idx, lp

        init = (
            jnp.int32(0),
            jnp.zeros((B, 1), jnp.float32),
            jnp.full((B, 1), -1, jnp.int32),
            jnp.zeros((B, 1), jnp.float32),
        )
        _, carry, idx, lp = jax.lax.while_loop(cond, body, init)

        def tail(args):
            carry, idx, lp = args
            copy = pltpu.make_async_copy(
                x_hbm.at[:, pl.ds(NCH * CS, TS)], tail_ref, sem
            )
            copy.start()
            copy.wait()
            return _process_chunk(tail_ref[...], NCH * CS, TS, carry, rand, idx, lp)

        carry, idx, lp = jax.lax.cond(
            jnp.all(carry >= rand), lambda a: a, tail, (carry, idx, lp)
        )
        idx_ref[...] = idx
        lp_ref[...] = lp

    col = i * CF + jax.lax.broadcasted_iota(jnp.int32, (B, CF), 1)
    out_ref[...] = jnp.where(col == idx_ref[...], 0.0, NEG_INF)


@jax.jit
def kernel(inputs, manualrand):
    out, lp = pl.pallas_call(
        _kernel,
        grid=(NBLK,),
        in_specs=[
            pl.BlockSpec(memory_space=pl.ANY),
            pl.BlockSpec((B, 1), lambda i: (0, 0)),
        ],
        out_specs=[
            pl.BlockSpec((B, CF), lambda i: (0, i)),
            pl.BlockSpec((B, 1), lambda i: (0, 0)),
        ],
        out_shape=[
            jax.ShapeDtypeStruct((B, V), jnp.float32),
            jax.ShapeDtypeStruct((B, 1), jnp.float32),
        ],
        scratch_shapes=[
            pltpu.VMEM((B, 1), jnp.int32),
            pltpu.VMEM((B, CS), jnp.float32),
            pltpu.VMEM((B, TS), jnp.float32),
            pltpu.SemaphoreType.DMA,
        ],
    )(inputs, manualrand)
    return out, lp


# speedup vs baseline: 5.8836x; 1.0247x over previous
import jax
import jax.numpy as jnp
from jax.experimental import pallas as pl
from jax.experimental.pallas import tpu as pltpu

B = 128
V = 100000
CS = 1024
NCH = V // CS
TS = V - NCH * CS
CF = 12800
NBLK = (V + CF - 1) // CF

NEG_INF = float("-inf")


def _cumsum_lanes(p, width):
    k = 1
    while k < width:
        shifted = jnp.concatenate(
            [jnp.zeros((p.shape[0], k), p.dtype), p[:, : width - k]], axis=1
        )
        p = p + shifted
        k *= 2
    return p


def _process_chunk(x, base, width, carry, rand, idx, lp):
    p = jnp.exp(x)
    total = carry + _cumsum_lanes(p, width)
    prev_total = jnp.concatenate([carry, total[:, : width - 1]], axis=1)
    onehot = jnp.logical_and(total >= rand, prev_total < rand)
    col = base + jax.lax.broadcasted_iota(jnp.int32, (B, width), 1)
    has = jnp.any(onehot, axis=1, keepdims=True)
    idx_new = jnp.sum(jnp.where(onehot, col, 0), axis=1, keepdims=True)
    idx = jnp.where(has, idx_new, idx)
    lp = lp + jnp.sum(jnp.where(onehot, x, 0.0), axis=1, keepdims=True)
    return total[:, width - 1 :], idx, lp


def _kernel(x_hbm, rand_ref, out_ref, lp_ref, idx_ref, chunk_ref, tail_ref, sem):
    i = pl.program_id(0)
    rand = rand_ref[...]

    @pl.when(i == 0)
    def _search():
        def cond(state):
            c, carry, _, _ = state
            return jnp.logical_and(c < NCH, jnp.logical_not(jnp.all(carry >= rand)))

        def body(state):
            c, carry, idx, lp = state
            copy = pltpu.make_async_copy(
                x_hbm.at[:, pl.ds(c * CS, CS)], chunk_ref, sem
            )
            copy.start()
            copy.wait()
            carry, idx, lp = _process_chunk(
                chunk_ref[...], c * CS, CS, carry, rand, idx, lp
            )
            return c + 1, carry, idx, lp

        init = (
            jnp.int32(0),
            jnp.zeros((B, 1), jnp.float32),
            jnp.full((B, 1), -1, jnp.int32),
            jnp.zeros((B, 1), jnp.float32),
        )
        _, carry, idx, lp = jax.lax.while_loop(cond, body, init)

        def tail(args):
            carry, idx, lp = args
            copy = pltpu.make_async_copy(
                x_hbm.at[:, pl.ds(NCH * CS, TS)], tail_ref, sem
            )
            copy.start()
            copy.wait()
            return _process_chunk(tail_ref[...], NCH * CS, TS, carry, rand, idx, lp)

        carry, idx, lp = jax.lax.cond(
            jnp.all(carry >= rand), lambda a: a, tail, (carry, idx, lp)
        )
        idx_ref[...] = idx
        lp_ref[...] = lp

    col = i * CF + jax.lax.broadcasted_iota(jnp.int32, (B, CF), 1)
    out_ref[...] = jnp.where(col == idx_ref[...], 0.0, NEG_INF)


@jax.jit
def kernel(inputs, manualrand):
    out, lp = pl.pallas_call(
        _kernel,
        grid=(NBLK,),
        in_specs=[
            pl.BlockSpec(memory_space=pl.ANY),
            pl.BlockSpec((B, 1), lambda i: (0, 0)),
        ],
        out_specs=[
            pl.BlockSpec((B, CF), lambda i: (0, i)),
            pl.BlockSpec((B, 1), lambda i: (0, 0)),
        ],
        out_shape=[
            jax.ShapeDtypeStruct((B, V), jnp.float32),
            jax.ShapeDtypeStruct((B, 1), jnp.float32),
        ],
        scratch_shapes=[
            pltpu.VMEM((B, 1), jnp.int32),
            pltpu.VMEM((B, CS), jnp.float32),
            pltpu.VMEM((B, TS), jnp.float32),
            pltpu.SemaphoreType.DMA,
        ],
    )(inputs, manualrand)
    return out, lp
